# Initial kernel scaffold; baseline (speedup 1.0000x reference)
import jax
import jax.numpy as jnp
from jax.experimental import pallas as pl


def kernel(x, edge1, pos, idx, ei2, node_feat, W1a, b1a, W1b, b1b, Wl2, bl2, ln_g, ln_b, W2, b2, W2r, b2r, Wp1, bp1, Wp2, bp2):
    raise NotImplementedError("write your pallas kernel here")


# speedup vs baseline: 33.4448x; 33.4448x over previous
import functools

import jax
import jax.numpy as jnp
from jax import lax
from jax.experimental import pallas as pl
from jax.experimental.pallas import tpu as pltpu
from jax.experimental.pallas import tpu_sc as plsc

EPS = 1e-5

NC = 2
NS = 16
NW = NC * NS

N = 10000
NPAD = 10240
DUMP = 10200
E1 = 160000
P = 65536
E2 = 1048576
I = 32768
D1 = 256
DH1 = 128
D2 = 32
DH2 = 16
CH = 128

_SEG1_NCH = 80
_SEG1_G = 2
_SEG2_NCH = 512
_SEG2_G = 8
_DEG1_NCH = 40
_DEG1_G = 8
_DEG2_NCH = 512
_DEG2_G = 32


def _mesh():
    return plsc.VectorSubcoreMesh(
        core_axis_name="c", subcore_axis_name="s", num_cores=NC,
        num_subcores=NS)


@functools.partial(
    pl.kernel,
    out_type=(jax.ShapeDtypeStruct((NC, NPAD), jnp.float32),
              jax.ShapeDtypeStruct((NC, P), jnp.float32)),
    mesh=_mesh(),
    compiler_params=pltpu.CompilerParams(use_tc_tiling_on_sc=False),
    scratch_types=[
        pltpu.VMEM((_DEG2_G, CH), jnp.float32),
        pltpu.VMEM((_DEG1_G, CH), jnp.int32),
        pltpu.VMEM((_DEG2_G, CH), jnp.int32),
        pltpu.SemaphoreType.DMA,
        pltpu.VMEM_SHARED((NPAD,), jnp.float32),
        pltpu.VMEM_SHARED((P,), jnp.float32),
    ],
)
def _sc_degrees(d1idx, d2idx, ones_hbm, zeros_hbm,
                deg1_out, deg2_out, ones_v, idx1_v, idx2_v, dsem, acc1,
                acc2):
    c = lax.axis_index("c")
    s = lax.axis_index("s")
    r1 = NPAD // NS
    r2 = P // NS
    pltpu.sync_copy(ones_hbm, ones_v)
    pltpu.sync_copy(zeros_hbm.at[pl.ds(0, r1)], acc1.at[pl.ds(s * r1, r1)])
    pltpu.sync_copy(zeros_hbm.at[pl.ds(0, r2)], acc2.at[pl.ds(s * r2, r2)])
    plsc.subcore_barrier()

    def count(idx_hbm, idx_v, acc, nch, grp):
        def body(g, carry):
            pltpu.sync_copy(idx_hbm.at[c, s, pl.ds(g * grp, grp)], idx_v)
            ds = [pltpu.async_copy(ones_v.at[j], acc.at[idx_v.at[j]],
                                   dsem, add=True) for j in range(grp)]
            for d in ds:
                d.wait()
            return carry

        lax.fori_loop(0, nch // grp, body, 0)

    count(d1idx, idx1_v, acc1, _DEG1_NCH, _DEG1_G)
    count(d2idx, idx2_v, acc2, _DEG2_NCH, _DEG2_G)
    plsc.subcore_barrier()
    pltpu.sync_copy(acc1.at[pl.ds(s * r1, r1)],
                    deg1_out.at[c, pl.ds(s * r1, r1)])
    pltpu.sync_copy(acc2.at[pl.ds(s * r2, r2)],
                    deg2_out.at[c, pl.ds(s * r2, r2)])


def _make_seg_sum(ppad, dh, nch, grp):
    rpt = ppad // NS

    @functools.partial(
        pl.kernel,
        out_type=jax.ShapeDtypeStruct((NC, ppad, dh), jnp.float32),
        mesh=_mesh(),
        compiler_params=pltpu.CompilerParams(use_tc_tiling_on_sc=False),
        scratch_types=[
            pltpu.VMEM((grp, CH), jnp.int32),
            pltpu.VMEM((grp, CH), jnp.int32),
            pltpu.VMEM((grp, CH, dh), jnp.float32),
            pltpu.SemaphoreType.DMA,
            pltpu.VMEM_SHARED((ppad, dh), jnp.float32),
        ],
    )
    def seg(table, sidx, didx, zeros_hbm, out, sidx_v, didx_v, buf, sem,
            acc):
        c = lax.axis_index("c")
        s = lax.axis_index("s")
        pltpu.sync_copy(zeros_hbm, acc.at[pl.ds(s * rpt, rpt)])
        plsc.subcore_barrier()

        def body(g, carry):
            pltpu.sync_copy(sidx.at[c, s, pl.ds(g * grp, grp)], sidx_v)
            pltpu.sync_copy(didx.at[s, pl.ds(g * grp, grp)], didx_v)
            ds = [pltpu.async_copy(table.at[sidx_v.at[j]], buf.at[j], sem)
                  for j in range(grp)]
            for d in ds:
                d.wait()
            ds = [pltpu.async_copy(buf.at[j], acc.at[didx_v.at[j]], sem,
                                   add=True) for j in range(grp)]
            for d in ds:
                d.wait()
            return carry

        lax.fori_loop(0, nch // grp, body, 0)
        plsc.subcore_barrier()
        pltpu.sync_copy(acc.at[pl.ds(s * rpt, rpt)],
                        out.at[c, pl.ds(s * rpt, rpt)])

    return seg


def _make_gather(nrows_tab, dh, npc, grp):
    @functools.partial(
        pl.kernel,
        out_type=jax.ShapeDtypeStruct((NW * npc, CH, dh), jnp.float32),
        mesh=_mesh(),
        compiler_params=pltpu.CompilerParams(use_tc_tiling_on_sc=False),
        scratch_types=[
            pltpu.VMEM((grp, CH), jnp.int32),
            pltpu.VMEM((grp, CH, dh), jnp.float32),
            pltpu.SemaphoreType.DMA,
        ],
    )
    def gat(table, idxs, out, idx_v, buf, sem):
        c = lax.axis_index("c")
        s = lax.axis_index("s")
        wid = c * NS + s

        def body(g, carry):
            pltpu.sync_copy(idxs.at[wid, pl.ds(g * grp, grp)], idx_v)
            ds = [pltpu.async_copy(table.at[idx_v.at[j]], buf.at[j], sem)
                  for j in range(grp)]
            for d in ds:
                d.wait()
            pltpu.sync_copy(buf, out.at[pl.ds(wid * npc + g * grp, grp)])
            return carry

        lax.fori_loop(0, npc // grp, body, 0)

    return gat


_seg1 = _make_seg_sum(NPAD, DH1, _SEG1_NCH, _SEG1_G)
_seg2 = _make_seg_sum(P, DH2, _SEG2_NCH, _SEG2_G)
_gather_pos = _make_gather(NPAD, D2, 32, 8)
_gather_fin = _make_gather(P, D2, 8, 8)


def _ln(v):
    m = jnp.mean(v, axis=-1, keepdims=True)
    var = jnp.mean((v - m) ** 2, axis=-1, keepdims=True)
    return (v - m) * lax.rsqrt(var + EPS)


def _dinv(degp, self1):
    d = degp[0] + degp[1] + self1
    return jnp.where(d > 0, lax.rsqrt(jnp.maximum(d, 1.0)), 0.0)


_NB1 = 10
_B1 = NPAD // _NB1
_NB2 = 32
_B2 = P // _NB2


def _tc_g1a_body(nf_ref, w_ref, degp_ref, self_ref, g_ref):
    dinv = _dinv(degp_ref[...], self_ref[...])
    g_ref[0] = jnp.dot(nf_ref[...], w_ref[...],
                       preferred_element_type=jnp.float32) * dinv


def _tc_g1a(nfpad, w1a, deg1p, self1):
    return pl.pallas_call(
        _tc_g1a_body,
        grid=(_NB1, NC),
        in_specs=[
            pl.BlockSpec((_B1, D1), lambda b, c: (b, 0)),
            pl.BlockSpec((D1, DH1), lambda b, c: (0, c)),
            pl.BlockSpec((NC, _B1, 1), lambda b, c: (0, b, 0)),
            pl.BlockSpec((_B1, 1), lambda b, c: (b, 0)),
        ],
        out_specs=pl.BlockSpec((1, _B1, DH1), lambda b, c: (c, b, 0)),
        out_shape=jax.ShapeDtypeStruct((NC, NPAD, DH1), jnp.float32),
    )(nfpad, w1a, deg1p, self1)


def _tc_comb1_body(agg_ref, g_ref, degp_ref, self_ref, b_ref, h0_ref,
                   wn_ref, h1_ref, gn_ref):
    dinv = _dinv(degp_ref[...], self_ref[...])
    agg = jnp.concatenate([agg_ref[0], agg_ref[1]], axis=-1)
    gg = jnp.concatenate([g_ref[0], g_ref[1]], axis=-1)
    t = (agg + gg) * dinv + b_ref[...]
    h1 = h0_ref[...] + jnp.maximum(_ln(t), 0.0)
    h1_ref[...] = h1
    gn_ref[0] = jnp.dot(h1, wn_ref[...],
                        preferred_element_type=jnp.float32) * dinv


def _tc_comb1(agg, g, deg1p, self1, bias, h0, wnext):
    return pl.pallas_call(
        _tc_comb1_body,
        grid=(_NB1, NC),
        in_specs=[
            pl.BlockSpec((NC, _B1, DH1), lambda b, c: (0, b, 0)),
            pl.BlockSpec((NC, _B1, DH1), lambda b, c: (0, b, 0)),
            pl.BlockSpec((NC, _B1, 1), lambda b, c: (0, b, 0)),
            pl.BlockSpec((_B1, 1), lambda b, c: (b, 0)),
            pl.BlockSpec((1, D1), lambda b, c: (0, 0)),
            pl.BlockSpec((_B1, D1), lambda b, c: (b, 0)),
            pl.BlockSpec((D1, DH1), lambda b, c: (0, c)),
        ],
        out_specs=[
            pl.BlockSpec((_B1, D1), lambda b, c: (b, 0)),
            pl.BlockSpec((1, _B1, DH1), lambda b, c: (c, b, 0)),
        ],
        out_shape=[
            jax.ShapeDtypeStruct((NPAD, D1), jnp.float32),
            jax.ShapeDtypeStruct((NC, NPAD, DH1), jnp.float32),
        ],
    )(agg, g, deg1p, self1, bias, h0, wnext)


def _tc_comb1b_body(agg_ref, g_ref, degp_ref, self_ref, b_ref, h1_ref,
                    wl2_ref, bl2_ref, lng_ref, lnb_ref, hs_ref):
    dinv = _dinv(degp_ref[...], self_ref[...])
    agg = jnp.concatenate([agg_ref[0], agg_ref[1]], axis=-1)
    gg = jnp.concatenate([g_ref[0], g_ref[1]], axis=-1)
    t = (agg + gg) * dinv + b_ref[...]
    h2 = h1_ref[...] + jnp.maximum(_ln(t), 0.0)
    hs = jnp.dot(h2, wl2_ref[...], preferred_element_type=jnp.float32)
    hs = hs + bl2_ref[...]
    hs_ref[...] = _ln(hs) * lng_ref[...] + lnb_ref[...]


def _tc_comb1b(agg, g, deg1p, self1, bias, h1, wl2, bl2, lng, lnb):
    return pl.pallas_call(
        _tc_comb1b_body,
        grid=(_NB1,),
        in_specs=[
            pl.BlockSpec((NC, _B1, DH1), lambda b: (0, b, 0)),
            pl.BlockSpec((NC, _B1, DH1), lambda b: (0, b, 0)),
            pl.BlockSpec((NC, _B1, 1), lambda b: (0, b, 0)),
            pl.BlockSpec((_B1, 1), lambda b: (b, 0)),
            pl.BlockSpec((1, D1), lambda b: (0, 0)),
            pl.BlockSpec((_B1, D1), lambda b: (b, 0)),
            pl.BlockSpec((D1, D2), lambda b: (0, 0)),
            pl.BlockSpec((1, D2), lambda b: (0, 0)),
            pl.BlockSpec((1, D2), lambda b: (0, 0)),
            pl.BlockSpec((1, D2), lambda b: (0, 0)),
        ],
        out_specs=pl.BlockSpec((_B1, D2), lambda b: (b, 0)),
        out_shape=jax.ShapeDtypeStruct((NPAD, D2), jnp.float32),
    )(agg, g, deg1p, self1, bias, h1, wl2, bl2, lng, lnb)


def _tc_pz_body(hlr_ref, degp_ref, w2_ref, w2r_ref, z_ref, gf_ref, gr_ref):
    z = hlr_ref[0] * hlr_ref[1]
    z_ref[...] = z
    dinvf = lax.rsqrt(degp_ref[0] + 1.0)
    dinvr = lax.rsqrt(degp_ref[1] + 1.0)
    gf_ref[0] = jnp.dot(z, w2_ref[0],
                        preferred_element_type=jnp.float32) * dinvf
    gr_ref[0] = jnp.dot(z, w2r_ref[0],
                        preferred_element_type=jnp.float32) * dinvr


def _tc_pz(hlr, deg2p, w2s, w2rs):
    return pl.pallas_call(
        _tc_pz_body,
        grid=(_NB2, NC),
        in_specs=[
            pl.BlockSpec((NC, _B2, D2), lambda b, c: (0, b, 0)),
            pl.BlockSpec((NC, _B2, 1), lambda b, c: (0, b, 0)),
            pl.BlockSpec((1, D2, DH2), lambda b, c: (c, 0, 0)),
            pl.BlockSpec((1, D2, DH2), lambda b, c: (c, 0, 0)),
        ],
        out_specs=[
            pl.BlockSpec((_B2, D2), lambda b, c: (b, 0)),
            pl.BlockSpec((1, _B2, DH2), lambda b, c: (c, b, 0)),
            pl.BlockSpec((1, _B2, DH2), lambda b, c: (c, b, 0)),
        ],
        out_shape=[
            jax.ShapeDtypeStruct((P, D2), jnp.float32),
            jax.ShapeDtypeStruct((NC, P, DH2), jnp.float32),
            jax.ShapeDtypeStruct((NC, P, DH2), jnp.float32),
        ],
    )(hlr, deg2p, w2s, w2rs)


def _tc_comb2_body(z_ref, aggf_ref, gf_ref, aggr_ref, gr_ref, degp_ref,
                   b2_ref, b2r_ref, z2_ref):
    dinvf = lax.rsqrt(degp_ref[0] + 1.0)
    dinvr = lax.rsqrt(degp_ref[1] + 1.0)
    aggf = jnp.concatenate([aggf_ref[0], aggf_ref[1]], axis=-1)
    gf = jnp.concatenate([gf_ref[0], gf_ref[1]], axis=-1)
    aggr = jnp.concatenate([aggr_ref[0], aggr_ref[1]], axis=-1)
    gr = jnp.concatenate([gr_ref[0], gr_ref[1]], axis=-1)
    tf = (aggf + gf) * dinvf + b2_ref[...]
    tr = (aggr + gr) * dinvr + b2r_ref[...]
    z2_ref[...] = (z_ref[...] + jnp.maximum(_ln(tf), 0.0)
                   + jnp.maximum(_ln(tr), 0.0))


def _tc_comb2(z, aggf, gf, aggr, gr, deg2p, b2, b2r):
    return pl.pallas_call(
        _tc_comb2_body,
        grid=(_NB2,),
        in_specs=[
            pl.BlockSpec((_B2, D2), lambda b: (b, 0)),
            pl.BlockSpec((NC, _B2, DH2), lambda b: (0, b, 0)),
            pl.BlockSpec((NC, _B2, DH2), lambda b: (0, b, 0)),
            pl.BlockSpec((NC, _B2, DH2), lambda b: (0, b, 0)),
            pl.BlockSpec((NC, _B2, DH2), lambda b: (0, b, 0)),
            pl.BlockSpec((NC, _B2, 1), lambda b: (0, b, 0)),
            pl.BlockSpec((1, D2), lambda b: (0, 0)),
            pl.BlockSpec((1, D2), lambda b: (0, 0)),
        ],
        out_specs=pl.BlockSpec((_B2, D2), lambda b: (b, 0)),
        out_shape=jax.ShapeDtypeStruct((P, D2), jnp.float32),
    )(z, aggf, gf, aggr, gr, deg2p, b2, b2r)


_NBF = 8
_BF = (I // 2) // _NBF


def _tc_final_body(zs_ref, wp1_ref, bp1_ref, wp2_ref, bp2_ref, o_ref):
    v = (zs_ref[:, :D2] + zs_ref[:, D2:]) * 0.5
    v = jnp.maximum(
        jnp.dot(v, wp1_ref[...], preferred_element_type=jnp.float32)
        + bp1_ref[...], 0.0)
    o_ref[...] = (jnp.dot(v, wp2_ref[...],
                          preferred_element_type=jnp.float32)
                  + bp2_ref[...])


def _tc_final(zs2, wp1, bp1, wp2, bp2):
    return pl.pallas_call(
        _tc_final_body,
        grid=(_NBF,),
        in_specs=[
            pl.BlockSpec((_BF, 2 * D2), lambda b: (b, 0)),
            pl.BlockSpec((D2, D2), lambda b: (0, 0)),
            pl.BlockSpec((1, D2), lambda b: (0, 0)),
            pl.BlockSpec((D2, 1), lambda b: (0, 0)),
            pl.BlockSpec((1, 1), lambda b: (0, 0)),
        ],
        out_specs=pl.BlockSpec((_BF, 1), lambda b: (b, 0)),
        out_shape=jax.ShapeDtypeStruct((I // 2, 1), jnp.float32),
    )(zs2, wp1, bp1, wp2, bp2)


def kernel(x, edge1, pos, idx, ei2, node_feat, W1a, b1a, W1b, b1b, Wl2,
           bl2, ln_g, ln_b, W2, b2, W2r, b2r, Wp1, bp1, Wp2, bp2):
    f32 = jnp.float32
    i32 = jnp.int32

    e1s, e1d = edge1[0], edge1[1]
    pad1 = NS * _SEG1_NCH * CH - E1
    spad = jnp.concatenate([e1s, jnp.full((pad1,), DUMP, i32)])
    dpad = jnp.concatenate([e1d, jnp.full((pad1,), DUMP, i32)])
    sidx1 = jnp.stack([spad, spad + NPAD]).reshape(NC, NS, _SEG1_NCH, CH)
    didx1 = dpad.reshape(NS, _SEG1_NCH, CH)
    d1idx = dpad.reshape(NC, NS, _DEG1_NCH, CH)

    s2, d2 = ei2[0], ei2[1]
    d2idx = jnp.stack([d2, s2]).reshape(NC, NS, _DEG2_NCH, CH)
    sidxF = jnp.stack([s2, s2 + P]).reshape(NC, NS, _SEG2_NCH, CH)
    didxF = d2.reshape(NS, _SEG2_NCH, CH)
    sidxR = jnp.stack([d2, d2 + P]).reshape(NC, NS, _SEG2_NCH, CH)
    didxR = s2.reshape(NS, _SEG2_NCH, CH)

    pos_idx = jnp.concatenate([pos[:, 0], pos[:, 1]]).reshape(NW, 32, CH)
    fin_idx = idx.reshape(NW, 8, CH)

    ones_c = jnp.ones((_DEG2_G, CH), f32)
    zeros_1d = jnp.zeros((P // NS,), f32)
    zeros_s1 = jnp.zeros((NPAD // NS, DH1), f32)
    zeros_s2 = jnp.zeros((P // NS, DH2), f32)
    self1 = (jnp.arange(NPAD) < N).astype(f32).reshape(NPAD, 1)
    nfpad = jnp.pad(node_feat, ((0, NPAD - N), (0, 0)))

    b1a2 = b1a.reshape(1, D1)
    b1b2 = b1b.reshape(1, D1)
    bl22 = bl2.reshape(1, D2)
    lng2 = ln_g.reshape(1, D2)
    lnb2 = ln_b.reshape(1, D2)
    b22 = b2.reshape(1, D2)
    b2r2 = b2r.reshape(1, D2)
    bp12 = bp1.reshape(1, D2)
    bp22 = bp2.reshape(1, 1)

    deg1p, deg2p = _sc_degrees(d1idx, d2idx, ones_c, zeros_1d)
    deg1p = deg1p.reshape(NC, NPAD, 1)
    deg2p = deg2p.reshape(NC, P, 1)

    g1a = _tc_g1a(nfpad, W1a, deg1p, self1)
    agg1a = _seg1(g1a.reshape(NC * NPAD, DH1), sidx1, didx1, zeros_s1)
    h1, g1b = _tc_comb1(agg1a, g1a, deg1p, self1, b1a2, nfpad, W1b)
    agg1b = _seg1(g1b.reshape(NC * NPAD, DH1), sidx1, didx1, zeros_s1)
    hsmall = _tc_comb1b(agg1b, g1b, deg1p, self1, b1b2, h1, Wl2, bl22,
                        lng2, lnb2)

    hlr = _gather_pos(hsmall, pos_idx).reshape(NC, P, D2)
    w2s = jnp.stack([W2[:, :DH2], W2[:, DH2:]])
    w2rs = jnp.stack([W2r[:, :DH2], W2r[:, DH2:]])
    z, gf, gr = _tc_pz(hlr, deg2p, w2s, w2rs)
    aggf = _seg2(gf.reshape(NC * P, DH2), sidxF, didxF, zeros_s2)
    aggr = _seg2(gr.reshape(NC * P, DH2), sidxR, didxR, zeros_s2)
    z2 = _tc_comb2(z, aggf, gf, aggr, gr, deg2p, b22, b2r2)

    zsel = _gather_fin(z2, fin_idx).reshape(I // 2, 2 * D2)
    return _tc_final(zsel, Wp1, bp12, Wp2, bp22)
